# Initial kernel scaffold; baseline (speedup 1.0000x reference)
import jax
import jax.numpy as jnp
from jax.experimental import pallas as pl


def kernel(x, weight):
    raise NotImplementedError("write your pallas kernel here")


# speedup vs baseline: 2.4271x; 2.4271x over previous
import math

import jax
import jax.numpy as jnp
from jax import lax
from jax.experimental import pallas as pl
from jax.experimental.pallas import tpu as pltpu
from jax.experimental.pallas import tpu_sc as plsc

VOCAB = 100000
D = 128
SCALE = math.sqrt(float(D))

NC = 2
NS = 16
NW = NC * NS
B_TOTAL = 4096 * 50
B_PER_W = B_TOTAL // NW
CHUNK = 128
N_CHUNKS = B_PER_W // CHUNK


def _sc_embed(table, idx3):
    mesh = plsc.VectorSubcoreMesh(core_axis_name="c", subcore_axis_name="s")

    def body(table_hbm, idx_hbm, out_hbm, idx_v, rows_v, gsem):
        wid = lax.axis_index("s") * NC + lax.axis_index("c")
        base = wid * B_PER_W
        pltpu.sync_copy(idx_hbm.at[wid], idx_v)

        def chunk_body(c, _):
            pltpu.async_copy(table_hbm.at[idx_v.at[c]], rows_v, gsem).wait()

            def row_body(r, _):
                for j in range(D // 16):
                    sl = pl.ds(j * 16, 16)
                    rows_v[r, sl] = rows_v[r, sl] * SCALE
                return _

            lax.fori_loop(0, CHUNK, row_body, 0, unroll=False)
            pltpu.sync_copy(
                rows_v, out_hbm.at[pl.ds(base + c * CHUNK, CHUNK)])
            return _

        lax.fori_loop(0, N_CHUNKS, chunk_body, 0, unroll=False)

    run = pl.kernel(
        body,
        out_type=jax.ShapeDtypeStruct((B_TOTAL, D), jnp.float32),
        mesh=mesh,
        scratch_types=[
            pltpu.VMEM((N_CHUNKS, CHUNK), jnp.int32),
            pltpu.VMEM((CHUNK, D), jnp.float32),
            pltpu.SemaphoreType.DMA,
        ],
    )
    return run(table, idx3)


def kernel(x, weight):
    idx3 = x.astype(jnp.int32).reshape(NW, N_CHUNKS, CHUNK)
    out = _sc_embed(weight, idx3)
    return out.reshape(x.shape[0], x.shape[1], D)


# speedup vs baseline: 2.9448x; 1.2133x over previous
import math

import jax
import jax.numpy as jnp
from jax import lax
from jax.experimental import pallas as pl
from jax.experimental.pallas import tpu as pltpu
from jax.experimental.pallas import tpu_sc as plsc

VOCAB = 100000
D = 128
SCALE = math.sqrt(float(D))

NC = 2
NS = 16
NW = NC * NS
B_TOTAL = 4096 * 50
B_PER_W = B_TOTAL // NW
CHUNK = 128
N_CHUNKS = B_PER_W // CHUNK
NB = 5
LA = 2


def _sc_embed(table, idx3):
    mesh = plsc.VectorSubcoreMesh(core_axis_name="c", subcore_axis_name="s")

    def body(table_hbm, idx_hbm, out_hbm, idx_v, bufs, *sems):
        gsem = sems[:NB]
        ssem = sems[NB:]
        wid = lax.axis_index("s") * NC + lax.axis_index("c")
        base = wid * B_PER_W
        pltpu.sync_copy(idx_hbm.at[wid], idx_v)

        def gather(c, b):
            pltpu.async_copy(
                table_hbm.at[idx_v.at[c]], bufs.at[b], gsem[b])

        def gather_wait(c, b):
            pltpu.make_async_copy(
                table_hbm.at[idx_v.at[c]], bufs.at[b], gsem[b]).wait()

        def scatter(c, b):
            pltpu.async_copy(
                bufs.at[b], out_hbm.at[pl.ds(base + c * CHUNK, CHUNK)],
                ssem[b])

        def scatter_wait(c, b):
            pltpu.make_async_copy(
                bufs.at[b], out_hbm.at[pl.ds(base + c * CHUNK, CHUNK)],
                ssem[b]).wait()

        def compute(c, b):
            gather_wait(c, b)

            def row_body(r, carry):
                for j in range(D // 16):
                    sl = pl.ds(j * 16, 16)
                    bufs[b, r, sl] = bufs[b, r, sl] * SCALE
                return carry

            lax.fori_loop(0, CHUNK, row_body, 0, unroll=False)
            scatter(c, b)

        for b in range(LA):
            gather(b, b)
        for db in range(NB):
            c2 = db + LA
            if c2 >= NB:
                scatter_wait(c2 - NB, c2 % NB)
            gather(c2, c2 % NB)
            compute(db, db)

        def group_body(i, carry):
            g = i * NB
            for db in range(NB):
                c = g + db
                b2 = (db + LA) % NB
                scatter_wait(c + LA - NB, b2)
                gather(c + LA, b2)
                compute(c, db)
            return carry

        lax.fori_loop(1, N_CHUNKS // NB - 1, group_body, 0, unroll=False)

        g = N_CHUNKS - NB
        for db in range(NB):
            c = g + db
            c2 = c + LA
            if c2 < N_CHUNKS:
                scatter_wait(c2 - NB, c2 % NB)
                gather(c2, c2 % NB)
            compute(c, db)
        for db in range(NB):
            scatter_wait(g + db, db)

    run = pl.kernel(
        body,
        out_type=jax.ShapeDtypeStruct((B_TOTAL, D), jnp.float32),
        mesh=mesh,
        scratch_types=(
            [pltpu.VMEM((N_CHUNKS, CHUNK), jnp.int32),
             pltpu.VMEM((NB, CHUNK, D), jnp.float32)]
            + [pltpu.SemaphoreType.DMA] * (2 * NB)
        ),
    )
    return run(table, idx3)


def kernel(x, weight):
    idx3 = x.astype(jnp.int32).reshape(NW, N_CHUNKS, CHUNK)
    out = _sc_embed(weight, idx3)
    return out.reshape(x.shape[0], x.shape[1], D)


# speedup vs baseline: 5.1004x; 1.7320x over previous
import math

import jax
import jax.numpy as jnp
from jax import lax
from jax.experimental import pallas as pl
from jax.experimental.pallas import tpu as pltpu
from jax.experimental.pallas import tpu_sc as plsc

VOCAB = 100000
D = 128
SCALE = math.sqrt(float(D))

NC = 2
NS = 16
NW = NC * NS
BATCH = 4096
SEQ = 50
B_PER_W = BATCH // NW
NB = 4
LA = 2


def _sc_embed(table, idx3):
    mesh = plsc.VectorSubcoreMesh(core_axis_name="c", subcore_axis_name="s")

    def body(table_hbm, idx_hbm, out_hbm, idx_v, bufs, *sems):
        gsem = sems[:NB]
        ssem = sems[NB:]
        wid = lax.axis_index("s") * NC + lax.axis_index("c")
        base = wid * B_PER_W
        pltpu.sync_copy(idx_hbm.at[wid], idx_v)

        def gather(c, b):
            pltpu.async_copy(
                table_hbm.at[idx_v.at[c]], bufs.at[b], gsem[b])

        def gather_wait(c, b):
            pltpu.make_async_copy(
                table_hbm.at[idx_v.at[c]], bufs.at[b], gsem[b]).wait()

        def scatter(c, b):
            pltpu.async_copy(bufs.at[b], out_hbm.at[base + c], ssem[b])

        def scatter_wait(c, b):
            pltpu.make_async_copy(
                bufs.at[b], out_hbm.at[base + c], ssem[b]).wait()

        def compute(c, b):
            gather_wait(c, b)

            def row_body(r, carry):
                for j in range(D // 16):
                    sl = pl.ds(j * 16, 16)
                    bufs[b, r, sl] = bufs[b, r, sl] * SCALE
                return carry

            lax.fori_loop(0, SEQ, row_body, 0, unroll=False)
            scatter(c, b)

        for b in range(LA):
            gather(b, b)
        for db in range(NB):
            c2 = db + LA
            if c2 >= NB:
                scatter_wait(c2 - NB, c2 % NB)
            gather(c2, c2 % NB)
            compute(db, db)

        def group_body(i, carry):
            g = i * NB
            for db in range(NB):
                c = g + db
                b2 = (db + LA) % NB
                scatter_wait(c + LA - NB, b2)
                gather(c + LA, b2)
                compute(c, db)
            return carry

        lax.fori_loop(1, B_PER_W // NB - 1, group_body, 0, unroll=False)

        g = B_PER_W - NB
        for db in range(NB):
            c = g + db
            c2 = c + LA
            if c2 < B_PER_W:
                scatter_wait(c2 - NB, c2 % NB)
                gather(c2, c2 % NB)
            compute(c, db)
        for db in range(NB):
            scatter_wait(g + db, db)

    run = pl.kernel(
        body,
        out_type=jax.ShapeDtypeStruct((BATCH, SEQ, D), jnp.float32),
        mesh=mesh,
        scratch_types=(
            [pltpu.VMEM((B_PER_W, SEQ), jnp.int32),
             pltpu.VMEM((NB, SEQ, D), jnp.float32)]
            + [pltpu.SemaphoreType.DMA] * (2 * NB)
        ),
    )
    return run(table, idx3)


def kernel(x, weight):
    idx3 = x.astype(jnp.int32).reshape(NW, B_PER_W, SEQ)
    return _sc_embed(weight, idx3)


# speedup vs baseline: 5.1205x; 1.0039x over previous
import math

import jax
import jax.numpy as jnp
from jax import lax
from jax.experimental import pallas as pl
from jax.experimental.pallas import tpu as pltpu
from jax.experimental.pallas import tpu_sc as plsc

VOCAB = 100000
D = 128
SCALE = math.sqrt(float(D))

NC = 2
NS = 16
NW = NC * NS
BATCH = 4096
SEQ = 50
B_PER_W = BATCH // NW
NB = 4
LA = 2


def _sc_embed(table, idx3):
    mesh = plsc.VectorSubcoreMesh(core_axis_name="c", subcore_axis_name="s")

    def body(table_hbm, idx_hbm, out_hbm, idx_v, bufs, *sems):
        gsem = sems[:NB]
        ssem = sems[NB:]
        wid = lax.axis_index("s") * NC + lax.axis_index("c")
        base = wid * B_PER_W
        pltpu.sync_copy(idx_hbm.at[wid], idx_v)

        def gather(c, b):
            pltpu.async_copy(
                table_hbm.at[idx_v.at[c]], bufs.at[b], gsem[b])

        def gather_wait(c, b):
            pltpu.make_async_copy(
                table_hbm.at[idx_v.at[c]], bufs.at[b], gsem[b]).wait()

        def scatter(c, b):
            pltpu.async_copy(bufs.at[b], out_hbm.at[base + c], ssem[b])

        def scatter_wait(c, b):
            pltpu.make_async_copy(
                bufs.at[b], out_hbm.at[base + c], ssem[b]).wait()

        def compute(c, b):
            gather_wait(c, b)

            def row_body(r, carry):
                for j in range(D // 16):
                    sl = pl.ds(j * 16, 16)
                    bufs[b, r, sl] = bufs[b, r, sl] * SCALE
                return carry

            lax.fori_loop(0, SEQ, row_body, 0, unroll=False)
            scatter(c, b)

        for b in range(LA):
            gather(b, b)
        for db in range(NB):
            c2 = db + LA
            if c2 >= NB:
                scatter_wait(c2 - NB, c2 % NB)
            gather(c2, c2 % NB)
            compute(db, db)

        def group_body(i, carry):
            g = i * NB
            for db in range(NB):
                c = g + db
                b2 = (db + LA) % NB
                scatter_wait(c + LA - NB, b2)
                gather(c + LA, b2)
                compute(c, db)
            return carry

        lax.fori_loop(1, B_PER_W // NB - 1, group_body, 0, unroll=False)

        g = B_PER_W - NB
        for db in range(NB):
            c = g + db
            c2 = c + LA
            if c2 < B_PER_W:
                scatter_wait(c2 - NB, c2 % NB)
                gather(c2, c2 % NB)
            compute(c, db)
        for db in range(NB):
            scatter_wait(g + db, db)

    run = pl.kernel(
        body,
        out_type=jax.ShapeDtypeStruct((BATCH, SEQ, D), jnp.float32),
        mesh=mesh,
        scratch_types=(
            [pltpu.VMEM((B_PER_W, SEQ), jnp.int32),
             pltpu.VMEM((NB, SEQ, D), jnp.float32)]
            + [pltpu.SemaphoreType.DMA] * (2 * NB)
        ),
        compiler_params=pltpu.CompilerParams(use_tc_tiling_on_sc=True),
    )
    return run(table, idx3)


def kernel(x, weight):
    idx3 = x.astype(jnp.int32).reshape(NW, B_PER_W, SEQ)
    return _sc_embed(weight, idx3)


# speedup vs baseline: 9.2625x; 1.8089x over previous
import math

import jax
import jax.numpy as jnp
from jax import lax
from jax.experimental import pallas as pl
from jax.experimental.pallas import tpu as pltpu
from jax.experimental.pallas import tpu_sc as plsc

VOCAB = 100000
D = 128
SCALE = math.sqrt(float(D))

NC = 2
NS = 16
NW = NC * NS
BATCH = 4096
SEQ = 50
B_PER_W = BATCH // NW
NB = 5
LA = 2


def _sc_embed(table, idx3):
    mesh = plsc.VectorSubcoreMesh(core_axis_name="c", subcore_axis_name="s")

    def body(table_hbm, idx_hbm, out_hbm, idx_v, bufs, *sems):
        gsem = sems[:NB]
        ssem = sems[NB:]
        wid = lax.axis_index("s") * NC + lax.axis_index("c")
        base = wid * B_PER_W
        pltpu.sync_copy(idx_hbm.at[wid], idx_v)

        def gather(c, b):
            pltpu.async_copy(
                table_hbm.at[idx_v.at[c]], bufs.at[b], gsem[b])

        def gather_wait(c, b):
            pltpu.make_async_copy(
                table_hbm.at[idx_v.at[c]], bufs.at[b], gsem[b]).wait()

        def scatter(c, b):
            pltpu.async_copy(
                bufs.at[b], out_hbm.at[c, pl.ds(base, B_PER_W)], ssem[b])

        def scatter_wait(c, b):
            pltpu.make_async_copy(
                bufs.at[b], out_hbm.at[c, pl.ds(base, B_PER_W)],
                ssem[b]).wait()

        def compute(c, b):
            gather_wait(c, b)

            def row_body(r, carry):
                for j in range(D // 16):
                    sl = pl.ds(j * 16, 16)
                    bufs[b, r, sl] = bufs[b, r, sl] * SCALE
                return carry

            lax.fori_loop(0, B_PER_W, row_body, 0, unroll=False)
            scatter(c, b)

        for b in range(LA):
            gather(b, b)
        for db in range(NB):
            c2 = db + LA
            if c2 >= NB:
                scatter_wait(c2 - NB, c2 % NB)
            gather(c2, c2 % NB)
            compute(db, db)

        def group_body(i, carry):
            g = i * NB
            for db in range(NB):
                c = g + db
                b2 = (db + LA) % NB
                scatter_wait(c + LA - NB, b2)
                gather(c + LA, b2)
                compute(c, db)
            return carry

        lax.fori_loop(1, SEQ // NB - 1, group_body, 0, unroll=False)

        g = SEQ - NB
        for db in range(NB):
            c = g + db
            c2 = c + LA
            if c2 < SEQ:
                scatter_wait(c2 - NB, c2 % NB)
                gather(c2, c2 % NB)
            compute(c, db)
        for db in range(NB):
            scatter_wait(g + db, db)

    run = pl.kernel(
        body,
        out_type=jax.ShapeDtypeStruct((SEQ, BATCH, D), jnp.float32),
        mesh=mesh,
        scratch_types=(
            [pltpu.VMEM((SEQ, B_PER_W), jnp.int32),
             pltpu.VMEM((NB, B_PER_W, D), jnp.float32)]
            + [pltpu.SemaphoreType.DMA] * (2 * NB)
        ),
        compiler_params=pltpu.CompilerParams(use_tc_tiling_on_sc=True),
    )
    return run(table, idx3)


def kernel(x, weight):
    idx3 = x.T.astype(jnp.int32).reshape(SEQ, NW, B_PER_W).transpose(1, 0, 2)
    out = _sc_embed(weight, idx3)
    return out.transpose(1, 0, 2)
